# baseline (device time: 92199 ns/iter reference)
import jax
import jax.numpy as jnp
from jax import lax
from jax.experimental import pallas as pl
from jax.experimental.pallas import tpu as pltpu

B = 32
H = 16
D = 128
BS = 32
NBT = 256
NPAGE = 256
T = NPAGE * BS
TC = 2048
NC = T // TC
NSTEP = NC * H
PACK = 256
NEG = -1e30
MCLAMP = -1e29


def _count_body(bt_ref, lens_ref, lc_ref):
    my_x = lax.axis_index("x")
    pid = my_x * NPAGE + lax.broadcasted_iota(jnp.int32, (B, NBT, NPAGE), 2)
    bt3 = bt_ref[...][:, :, None]
    jidx = lax.broadcasted_iota(jnp.int32, (B, NBT, NPAGE), 1)
    valid = jidx < lens_ref[...][:, :, None]
    cnt_pages = jnp.sum(
        jnp.where((bt3 == pid) & valid, 1.0, 0.0), axis=1
    )
    lc_pages = jnp.where(cnt_pages > 0.0, jnp.log(cnt_pages), NEG)
    tp = lax.broadcasted_iota(jnp.int32, (NPAGE, T), 1) // BS
    pp = lax.broadcasted_iota(jnp.int32, (NPAGE, T), 0)
    expand = jnp.where(tp == pp, 1.0, 0.0)
    lc_ref[...] = jnp.dot(
        lc_pages, expand, preferred_element_type=jnp.float32
    )


def _partial_body(q_ref, k_ref, v_ref, lc_ref, part_ref,
                  qt_ref, kbuf, vbuf, ksem, vsem):
    c = pl.program_id(0)
    h = pl.program_id(1)
    n = c * H + h

    def k_dma(step, slot):
        cc = step // H
        hh = step % H
        return pltpu.make_async_copy(
            k_ref.at[pl.ds(cc * TC, TC), hh], kbuf.at[slot], ksem.at[slot]
        )

    def v_dma(step, slot):
        cc = step // H
        hh = step % H
        return pltpu.make_async_copy(
            v_ref.at[pl.ds(cc * TC, TC), hh], vbuf.at[slot], vsem.at[slot]
        )

    @pl.when(n == 0)
    def _():
        qt_ref[...] = jnp.swapaxes(
            (q_ref[...] * (D ** -0.5)).astype(jnp.bfloat16), 0, 1
        )
        part_ref[...] = jnp.zeros((H, B, PACK), jnp.float32)
        part_ref[:, :, D:D + 1] = jnp.full((H, B, 1), MCLAMP, jnp.float32)
        k_dma(0, 0).start()
        v_dma(0, 0).start()
        k_dma(1, 1).start()
        v_dma(1, 1).start()

    slot = lax.rem(n, 2)
    k_dma(n, slot).wait()
    v_dma(n, slot).wait()

    k = kbuf[slot].astype(jnp.bfloat16)
    v = vbuf[slot].astype(jnp.bfloat16)
    q = qt_ref[h]
    lc = lc_ref[:, pl.ds(c * TC, TC)]
    s = lax.dot_general(
        q, k, (((1,), (1,)), ((), ())),
        preferred_element_type=jnp.float32,
    ) + lc

    m_old = part_ref[h, :, D:D + 1]
    l_old = part_ref[h, :, D + 1:D + 2]
    o_old = part_ref[h, :, 0:D]
    m_new = jnp.maximum(m_old, jnp.max(s, axis=1, keepdims=True))
    p = jnp.exp(s - m_new)
    scale = jnp.exp(m_old - m_new)
    l_new = l_old * scale + jnp.sum(p, axis=1, keepdims=True)
    o_new = o_old * scale + lax.dot_general(
        p.astype(jnp.bfloat16), v, (((1,), (0,)), ((), ())),
        preferred_element_type=jnp.float32,
    )
    part_ref[h, :, 0:D] = o_new
    part_ref[h, :, D:D + 1] = m_new
    part_ref[h, :, D + 1:D + 2] = l_new

    @pl.when(n + 2 < NSTEP)
    def _():
        nslot = slot
        k_dma(n + 2, nslot).start()
        v_dma(n + 2, nslot).start()


def _combine_body(part_ref, out_ref, rx_ref, send_sem, recv_sem):
    my_x = lax.axis_index("x")
    my_y = lax.axis_index("y")
    my_z = lax.axis_index("z")
    peer = (1 - my_x, my_y, my_z)

    bar = pltpu.get_barrier_semaphore()
    pl.semaphore_signal(
        bar, inc=1, device_id=peer, device_id_type=pl.DeviceIdType.MESH
    )
    pl.semaphore_wait(bar, 1)

    rdma = pltpu.make_async_remote_copy(
        src_ref=part_ref,
        dst_ref=rx_ref,
        send_sem=send_sem,
        recv_sem=recv_sem,
        device_id=peer,
        device_id_type=pl.DeviceIdType.MESH,
    )
    rdma.start()
    rdma.wait()

    o_l = part_ref[:, :, 0:D]
    m_l = part_ref[:, :, D:D + 1]
    l_l = part_ref[:, :, D + 1:D + 2]
    o_r = rx_ref[:, :, 0:D]
    m_r = rx_ref[:, :, D:D + 1]
    l_r = rx_ref[:, :, D + 1:D + 2]

    m_c = jnp.maximum(m_l, m_r)
    a_l = jnp.exp(m_l - m_c)
    a_r = jnp.exp(m_r - m_c)
    l_c = l_l * a_l + l_r * a_r
    o = (o_l * a_l + o_r * a_r) / l_c
    for hh in range(H):
        out_ref[:, 0, hh, :] = o[hh]


def kernel(Q, K, V, bt, lens):
    lens2 = lens.reshape(B, 1)
    Q2 = Q.reshape(B, H, D)
    K2 = K.reshape(T, H, D)
    V2 = V.reshape(T, H, D)

    lc_tok = pl.pallas_call(
        _count_body,
        in_specs=[
            pl.BlockSpec(memory_space=pltpu.VMEM),
            pl.BlockSpec(memory_space=pltpu.VMEM),
        ],
        out_specs=pl.BlockSpec(memory_space=pltpu.VMEM),
        out_shape=jax.ShapeDtypeStruct((B, T), jnp.float32),
    )(bt, lens2)

    part = pl.pallas_call(
        _partial_body,
        grid=(NC, H),
        in_specs=[
            pl.BlockSpec(memory_space=pltpu.VMEM),
            pl.BlockSpec(memory_space=pl.ANY),
            pl.BlockSpec(memory_space=pl.ANY),
            pl.BlockSpec(memory_space=pltpu.VMEM),
        ],
        out_specs=pl.BlockSpec((H, B, PACK), lambda c, h: (0, 0, 0)),
        out_shape=jax.ShapeDtypeStruct((H, B, PACK), jnp.float32),
        scratch_shapes=[
            pltpu.VMEM((H, B, D), jnp.bfloat16),
            pltpu.VMEM((2, TC, D), jnp.float32),
            pltpu.VMEM((2, TC, D), jnp.float32),
            pltpu.SemaphoreType.DMA((2,)),
            pltpu.SemaphoreType.DMA((2,)),
        ],
        compiler_params=pltpu.CompilerParams(
            vmem_limit_bytes=60 * 1024 * 1024
        ),
    )(Q2, K2, V2, lc_tok)

    return pl.pallas_call(
        _combine_body,
        out_shape=jax.ShapeDtypeStruct((B, 1, H, D), jnp.float32),
        in_specs=[pl.BlockSpec(memory_space=pltpu.VMEM)],
        out_specs=pl.BlockSpec(memory_space=pltpu.VMEM),
        scratch_shapes=[
            pltpu.VMEM((H, B, PACK), jnp.float32),
            pltpu.SemaphoreType.DMA,
            pltpu.SemaphoreType.DMA,
        ],
        compiler_params=pltpu.CompilerParams(collective_id=0),
    )(part)
